# baseline (speedup 1.0000x reference)
import functools

import jax
import jax.numpy as jnp
from jax import lax
from jax.experimental import pallas as pl
from jax.experimental.pallas import tpu as pltpu
from jax.experimental.pallas import tpu_sc as plsc

COUNT = 6
EPS = 1e-05
NUM_CLASSES = 19
C_PAD = 24

NW = 32
DL = 16
CH = 4096
L = 16


def _sc_body(x_hbm, y_hbm, sum_out, sq_out, cnt_out,
             xbuf, lbuf, sum_acc, sq_acc, cnt_acc, fin_s, fin_q, fin_c):
    wid = lax.axis_index("s") * 2 + lax.axis_index("c")
    b = wid // 4
    q = wid % 4
    d0 = q * DL
    N = x_hbm.shape[2]
    nch = N // CH

    zeros = jnp.zeros((L,), jnp.float32)
    ones = jnp.ones((L,), jnp.float32)
    lane = lax.broadcasted_iota(jnp.int32, (L,), 0)

    def init_body(i, _):
        sum_acc[pl.ds(i * L, L)] = zeros
        sq_acc[pl.ds(i * L, L)] = zeros
        return 0

    lax.fori_loop(0, DL * NUM_CLASSES, init_body, 0)

    def init_cnt(i, _):
        cnt_acc[pl.ds(i * L, L)] = zeros
        return 0

    lax.fori_loop(0, 2 * L, init_cnt, 0)

    def chunk_body(t, _):
        pltpu.sync_copy(x_hbm.at[b, pl.ds(d0, DL), pl.ds(t * CH, CH)], xbuf)
        pltpu.sync_copy(y_hbm.at[b, pl.ds(t * CH, CH)], lbuf)

        def group_body(g, _):
            lv = lbuf[pl.ds(g * L, L)]
            idxc = lv * L + lane
            plsc.addupdate_scatter(cnt_acc, [idxc], ones)
            for dl in range(DL):
                v = xbuf[dl, pl.ds(g * L, L)]
                idx = idxc + dl * (NUM_CLASSES * L)
                plsc.addupdate_scatter(sum_acc, [idx], v)
                plsc.addupdate_scatter(sq_acc, [idx], v * v)
            return 0

        lax.fori_loop(0, CH // L, group_body, 0)
        return 0

    lax.fori_loop(0, nch, chunk_body, 0)

    lane_row = lane * (NUM_CLASSES * L)
    for c in range(C_PAD):
        if c < NUM_CLASSES:
            vs = zeros
            vq = zeros
            for l in range(L):
                idx = lane_row + (c * L + l)
                vs = vs + plsc.load_gather(sum_acc, [idx])
                vq = vq + plsc.load_gather(sq_acc, [idx])
            fin_s[c, :] = vs
            fin_q[c, :] = vq
        else:
            fin_s[c, :] = zeros
            fin_q[c, :] = zeros
    for h in range(2):
        vc = zeros
        for l in range(L):
            idx = (h * L + lane) * L + l
            vc = vc + plsc.load_gather(cnt_acc, [idx])
        fin_c[pl.ds(h * L, L)] = vc

    pltpu.sync_copy(fin_s, sum_out.at[b, q])
    pltpu.sync_copy(fin_q, sq_out.at[b, q])

    @pl.when(q == 0)
    def _():
        pltpu.sync_copy(fin_c, cnt_out.at[b])


def _sc_segsum(x, y):
    B, d, N = x.shape
    mesh = plsc.VectorSubcoreMesh(core_axis_name="c", subcore_axis_name="s")
    f = pl.kernel(
        _sc_body,
        out_type=[
            jax.ShapeDtypeStruct((B, 4, C_PAD, DL), jnp.float32),
            jax.ShapeDtypeStruct((B, 4, C_PAD, DL), jnp.float32),
            jax.ShapeDtypeStruct((B, 2 * L), jnp.float32),
        ],
        mesh=mesh,
        scratch_types=[
            pltpu.VMEM((DL, CH), jnp.float32),
            pltpu.VMEM((CH,), jnp.int32),
            pltpu.VMEM((DL * NUM_CLASSES * L,), jnp.float32),
            pltpu.VMEM((DL * NUM_CLASSES * L,), jnp.float32),
            pltpu.VMEM((2 * L * L,), jnp.float32),
            pltpu.VMEM((C_PAD, DL), jnp.float32),
            pltpu.VMEM((C_PAD, DL), jnp.float32),
            pltpu.VMEM((2 * L,), jnp.float32),
        ],
        compiler_params=pltpu.CompilerParams(needs_layout_passes=False),
    )
    return f(x, y)


def _fin_body(s_ref, q_ref, c_ref, mean_ref, std_ref, valid_ref):
    s4 = s_ref[0]
    q4 = q_ref[0]
    s = jnp.concatenate([s4[i] for i in range(4)], axis=-1)
    s2 = jnp.concatenate([q4[i] for i in range(4)], axis=-1)
    cnt = c_ref[0]
    safe = jnp.maximum(cnt, 1.0)
    mean = s / safe
    denom = jnp.maximum(cnt - 1.0, 1.0)
    var = jnp.maximum((s2 - safe * mean * mean) / denom, 0.0)
    std = jnp.sqrt(var) + EPS
    v = cnt > float(COUNT)
    mean_ref[0] = jnp.where(v, mean, 0.0)
    std_ref[0] = jnp.where(v, std, 0.0)
    valid_ref[0] = v.astype(jnp.float32)


def kernel(x, y):
    B, d, N = x.shape
    sums, sqs, cnts = _sc_segsum(x, y)
    out_mean, out_std, out_valid = pl.pallas_call(
        _fin_body,
        grid=(B,),
        in_specs=[
            pl.BlockSpec((1, 4, C_PAD, DL), lambda b: (b, 0, 0, 0)),
            pl.BlockSpec((1, 4, C_PAD, DL), lambda b: (b, 0, 0, 0)),
            pl.BlockSpec((1, C_PAD, 1), lambda b: (b, 0, 0)),
        ],
        out_specs=[
            pl.BlockSpec((1, C_PAD, d), lambda b: (b, 0, 0)),
            pl.BlockSpec((1, C_PAD, d), lambda b: (b, 0, 0)),
            pl.BlockSpec((1, C_PAD, 1), lambda b: (b, 0, 0)),
        ],
        out_shape=[
            jax.ShapeDtypeStruct((B, C_PAD, d), jnp.float32),
            jax.ShapeDtypeStruct((B, C_PAD, d), jnp.float32),
            jax.ShapeDtypeStruct((B, C_PAD, 1), jnp.float32),
        ],
    )(sums, sqs, cnts[:, :C_PAD, None])
    means = out_mean[:, :NUM_CLASSES, :]
    stds = out_std[:, :NUM_CLASSES, :]
    valid = out_valid[:, :NUM_CLASSES, 0] > 0.5
    return (means, stds, valid)


# speedup vs baseline: 4.7217x; 4.7217x over previous
import jax
import jax.numpy as jnp
from jax import lax
from jax.experimental import pallas as pl
from jax.experimental.pallas import tpu as pltpu
from jax.experimental.pallas import tpu_sc as plsc

COUNT = 6
EPS = 1e-05
NUM_CLASSES = 19
C_PAD = 24

D_TC = 56
D_SC = 8
NP = 4
CH = 8192
L = 16


def _tc_body(x_ref, y_ref, sum_ref, sq_ref, cnt_ref):
    xb = x_ref[0]
    lab = y_ref[0]
    cls = jax.lax.broadcasted_iota(jnp.int32, (C_PAD, xb.shape[1]), 0)
    oh = (cls == lab).astype(jnp.float32)
    dn = (((1,), (1,)), ((), ()))
    sum_ref[0] = jax.lax.dot_general(oh, xb, dn, preferred_element_type=jnp.float32)
    sq_ref[0] = jax.lax.dot_general(oh, xb * xb, dn, preferred_element_type=jnp.float32)
    cnt_ref[0] = jnp.sum(oh, axis=1, keepdims=True)


def _tc_partial(x, y):
    B, d, N = x.shape
    y3 = y.reshape(B, 1, N)
    return pl.pallas_call(
        _tc_body,
        grid=(B,),
        in_specs=[
            pl.BlockSpec((1, D_TC, N), lambda b: (b, 0, 0)),
            pl.BlockSpec((1, 1, N), lambda b: (b, 0, 0)),
        ],
        out_specs=[
            pl.BlockSpec((1, C_PAD, D_TC), lambda b: (b, 0, 0)),
            pl.BlockSpec((1, C_PAD, D_TC), lambda b: (b, 0, 0)),
            pl.BlockSpec((1, C_PAD, 1), lambda b: (b, 0, 0)),
        ],
        out_shape=[
            jax.ShapeDtypeStruct((B, C_PAD, D_TC), jnp.float32),
            jax.ShapeDtypeStruct((B, C_PAD, D_TC), jnp.float32),
            jax.ShapeDtypeStruct((B, C_PAD, 1), jnp.float32),
        ],
    )(x, y3)


def _sc_body(x_hbm, y_hbm, sum_out, sq_out,
             xbuf, lbuf, sum_acc, sq_acc, fin_s, fin_q):
    wid = lax.axis_index("s") * 2 + lax.axis_index("c")
    b = wid // NP
    p = wid % NP
    N = x_hbm.shape[2]
    npix = N // NP
    base = p * npix
    nch = npix // CH

    zeros = jnp.zeros((L,), jnp.float32)
    lane = lax.broadcasted_iota(jnp.int32, (L,), 0)
    row = NUM_CLASSES * L

    def init_body(i, _):
        sum_acc[pl.ds(i * L, L)] = zeros
        sq_acc[pl.ds(i * L, L)] = zeros
        return 0

    lax.fori_loop(0, L * NUM_CLASSES, init_body, 0)

    def chunk_body(t, _):
        pltpu.sync_copy(
            x_hbm.at[b, pl.ds(D_TC, D_SC), pl.ds(base + t * CH, CH)], xbuf)
        pltpu.sync_copy(y_hbm.at[b, pl.ds(base + t * CH, CH)], lbuf)

        def group_body(g, _):
            lv = lbuf[pl.ds(g * L, L)]
            idxc = lv * L + lane
            for dl in range(D_SC):
                v = xbuf[dl, pl.ds(g * L, L)]
                idx = idxc + dl * row
                plsc.addupdate_scatter(sum_acc, [idx], v)
                plsc.addupdate_scatter(sq_acc, [idx], v * v)
            return 0

        lax.fori_loop(0, CH // L, group_body, 0)
        return 0

    lax.fori_loop(0, nch, chunk_body, 0)

    lane_row = lane * row
    for c in range(C_PAD):
        if c < NUM_CLASSES:
            vs = zeros
            vq = zeros
            for l in range(L):
                idx = lane_row + (c * L + l)
                vs = vs + plsc.load_gather(sum_acc, [idx])
                vq = vq + plsc.load_gather(sq_acc, [idx])
            fin_s[c, :] = vs
            fin_q[c, :] = vq
        else:
            fin_s[c, :] = zeros
            fin_q[c, :] = zeros

    pltpu.sync_copy(fin_s, sum_out.at[b, p])
    pltpu.sync_copy(fin_q, sq_out.at[b, p])


def _sc_partial(x, y):
    B, d, N = x.shape
    mesh = plsc.VectorSubcoreMesh(core_axis_name="c", subcore_axis_name="s")
    f = pl.kernel(
        _sc_body,
        out_type=[
            jax.ShapeDtypeStruct((B, NP, C_PAD, L), jnp.float32),
            jax.ShapeDtypeStruct((B, NP, C_PAD, L), jnp.float32),
        ],
        mesh=mesh,
        scratch_types=[
            pltpu.VMEM((D_SC, CH), jnp.float32),
            pltpu.VMEM((CH,), jnp.int32),
            pltpu.VMEM((L * NUM_CLASSES * L,), jnp.float32),
            pltpu.VMEM((L * NUM_CLASSES * L,), jnp.float32),
            pltpu.VMEM((C_PAD, L), jnp.float32),
            pltpu.VMEM((C_PAD, L), jnp.float32),
        ],
        compiler_params=pltpu.CompilerParams(needs_layout_passes=False),
    )
    return f(x, y)


def _fin_body(ts_ref, tq_ref, c_ref, ss_ref, sq_ref, mean_ref, std_ref, valid_ref):
    ssc = jnp.sum(ss_ref[0], axis=0)[:, :D_SC]
    qsc = jnp.sum(sq_ref[0], axis=0)[:, :D_SC]
    s = jnp.concatenate([ts_ref[0], ssc], axis=-1)
    s2 = jnp.concatenate([tq_ref[0], qsc], axis=-1)
    cnt = c_ref[0]
    safe = jnp.maximum(cnt, 1.0)
    mean = s / safe
    denom = jnp.maximum(cnt - 1.0, 1.0)
    var = jnp.maximum((s2 - safe * mean * mean) / denom, 0.0)
    std = jnp.sqrt(var) + EPS
    v = cnt > float(COUNT)
    mean_ref[0] = jnp.where(v, mean, 0.0)
    std_ref[0] = jnp.where(v, std, 0.0)
    valid_ref[0] = v.astype(jnp.float32)


def kernel(x, y):
    B, d, N = x.shape
    tc_s, tc_q, tc_c = _tc_partial(x, y)
    sc_s, sc_q = _sc_partial(x, y)
    out_mean, out_std, out_valid = pl.pallas_call(
        _fin_body,
        grid=(B,),
        in_specs=[
            pl.BlockSpec((1, C_PAD, D_TC), lambda b: (b, 0, 0)),
            pl.BlockSpec((1, C_PAD, D_TC), lambda b: (b, 0, 0)),
            pl.BlockSpec((1, C_PAD, 1), lambda b: (b, 0, 0)),
            pl.BlockSpec((1, NP, C_PAD, L), lambda b: (b, 0, 0, 0)),
            pl.BlockSpec((1, NP, C_PAD, L), lambda b: (b, 0, 0, 0)),
        ],
        out_specs=[
            pl.BlockSpec((1, C_PAD, d), lambda b: (b, 0, 0)),
            pl.BlockSpec((1, C_PAD, d), lambda b: (b, 0, 0)),
            pl.BlockSpec((1, C_PAD, 1), lambda b: (b, 0, 0)),
        ],
        out_shape=[
            jax.ShapeDtypeStruct((B, C_PAD, d), jnp.float32),
            jax.ShapeDtypeStruct((B, C_PAD, d), jnp.float32),
            jax.ShapeDtypeStruct((B, C_PAD, 1), jnp.float32),
        ],
    )(tc_s, tc_q, tc_c, sc_s, sc_q)
    means = out_mean[:, :NUM_CLASSES, :]
    stds = out_std[:, :NUM_CLASSES, :]
    valid = out_valid[:, :NUM_CLASSES, 0] > 0.5
    return (means, stds, valid)


# speedup vs baseline: 5.8301x; 1.2347x over previous
import jax
import jax.numpy as jnp
from jax import lax
from jax.experimental import pallas as pl
from jax.experimental.pallas import tpu as pltpu
from jax.experimental.pallas import tpu_sc as plsc

COUNT = 6
EPS = 1e-05
NUM_CLASSES = 19
C_PAD = 24

D_TC = 56
D_SC = 8
NP = 4
CH = 4096
L = 16
ROW = NUM_CLASSES * L


def _tc_body(x_ref, y_ref, sum_ref, sq_ref, cnt_ref):
    b_i = pl.program_id(0)
    xb = x_ref[0]
    lab = y_ref[pl.ds(b_i, 1), :]
    cls = jax.lax.broadcasted_iota(jnp.int32, (C_PAD, xb.shape[1]), 0)
    oh = (cls == lab).astype(jnp.float32)
    dn = (((1,), (1,)), ((), ()))
    sum_ref[0] = jax.lax.dot_general(oh, xb, dn, preferred_element_type=jnp.float32)
    sq_ref[0] = jax.lax.dot_general(oh, xb * xb, dn, preferred_element_type=jnp.float32)
    cnt_ref[0] = jnp.sum(oh, axis=1, keepdims=True)


def _tc_partial(x, y):
    B, d, N = x.shape
    return pl.pallas_call(
        _tc_body,
        grid=(B,),
        in_specs=[
            pl.BlockSpec((1, D_TC, N), lambda b: (b, 0, 0)),
            pl.BlockSpec((B, N), lambda b: (0, 0)),
        ],
        out_specs=[
            pl.BlockSpec((1, C_PAD, D_TC), lambda b: (b, 0, 0)),
            pl.BlockSpec((1, C_PAD, D_TC), lambda b: (b, 0, 0)),
            pl.BlockSpec((1, C_PAD, 1), lambda b: (b, 0, 0)),
        ],
        out_shape=[
            jax.ShapeDtypeStruct((B, C_PAD, D_TC), jnp.float32),
            jax.ShapeDtypeStruct((B, C_PAD, D_TC), jnp.float32),
            jax.ShapeDtypeStruct((B, C_PAD, 1), jnp.float32),
        ],
    )(x, y)


def _sc_body(x_hbm, y_hbm, sum_out, sq_out,
             xb0, xb1, lb0, lb1, sum_e, sum_o, sq_e, sq_o, fin_s, fin_q,
             sx0, sx1, sy0, sy1):
    wid = lax.axis_index("s") * 2 + lax.axis_index("c")
    b = wid // NP
    p = wid % NP
    N = x_hbm.shape[2]
    npix = N // NP
    base = p * npix
    nch = npix // CH

    zeros = jnp.zeros((L,), jnp.float32)
    lane = lax.broadcasted_iota(jnp.int32, (L,), 0)

    def init_body(i, _):
        sum_e[pl.ds(i * L, L)] = zeros
        sum_o[pl.ds(i * L, L)] = zeros
        sq_e[pl.ds(i * L, L)] = zeros
        sq_o[pl.ds(i * L, L)] = zeros
        return 0

    lax.fori_loop(0, L * NUM_CLASSES, init_body, 0)

    xbufs = (xb0, xb1)
    lbufs = (lb0, lb1)
    xsems = (sx0, sx1)
    ysems = (sy0, sy1)

    def start(t):
        s = t % 2
        hx = pltpu.async_copy(
            x_hbm.at[b, pl.ds(D_TC, D_SC), pl.ds(base + t * CH, CH)],
            xbufs[s], xsems[s])
        hy = pltpu.async_copy(
            y_hbm.at[b, pl.ds(base + t * CH, CH)], lbufs[s], ysems[s])
        return hx, hy

    hands = start(0)
    for t in range(nch):
        s = t % 2
        nxt = start(t + 1) if t + 1 < nch else None
        hands[0].wait()
        hands[1].wait()
        xbuf = xbufs[s]
        lbuf = lbufs[s]

        def group_body(i, _):
            g0 = i * 2
            lv0 = lbuf[pl.ds(g0 * L, L)]
            lv1 = lbuf[pl.ds(g0 * L + L, L)]
            idx0 = lv0 * L + lane
            idx1 = lv1 * L + lane
            for dl in range(D_SC):
                v0 = xbuf[dl, pl.ds(g0 * L, L)]
                v1 = xbuf[dl, pl.ds(g0 * L + L, L)]
                ie = idx0 + dl * ROW
                io = idx1 + dl * ROW
                plsc.addupdate_scatter(sum_e, [ie], v0)
                plsc.addupdate_scatter(sum_o, [io], v1)
                plsc.addupdate_scatter(sq_e, [ie], v0 * v0)
                plsc.addupdate_scatter(sq_o, [io], v1 * v1)
            return 0

        lax.fori_loop(0, CH // (2 * L), group_body, 0)
        hands = nxt

    lane_row = lane * ROW
    for c in range(C_PAD):
        if c < NUM_CLASSES:
            vs = zeros
            vq = zeros
            for l in range(L):
                idx = lane_row + (c * L + l)
                vs = vs + plsc.load_gather(sum_e, [idx]) + plsc.load_gather(sum_o, [idx])
                vq = vq + plsc.load_gather(sq_e, [idx]) + plsc.load_gather(sq_o, [idx])
            fin_s[c, :] = vs
            fin_q[c, :] = vq
        else:
            fin_s[c, :] = zeros
            fin_q[c, :] = zeros

    pltpu.sync_copy(fin_s, sum_out.at[b, p])
    pltpu.sync_copy(fin_q, sq_out.at[b, p])


def _sc_partial(x, y):
    B, d, N = x.shape
    mesh = plsc.VectorSubcoreMesh(core_axis_name="c", subcore_axis_name="s")
    f = pl.kernel(
        _sc_body,
        out_type=[
            jax.ShapeDtypeStruct((B, NP, C_PAD, L), jnp.float32),
            jax.ShapeDtypeStruct((B, NP, C_PAD, L), jnp.float32),
        ],
        mesh=mesh,
        scratch_types=[
            pltpu.VMEM((D_SC, CH), jnp.float32),
            pltpu.VMEM((D_SC, CH), jnp.float32),
            pltpu.VMEM((CH,), jnp.int32),
            pltpu.VMEM((CH,), jnp.int32),
            pltpu.VMEM((L * NUM_CLASSES * L,), jnp.float32),
            pltpu.VMEM((L * NUM_CLASSES * L,), jnp.float32),
            pltpu.VMEM((L * NUM_CLASSES * L,), jnp.float32),
            pltpu.VMEM((L * NUM_CLASSES * L,), jnp.float32),
            pltpu.VMEM((C_PAD, L), jnp.float32),
            pltpu.VMEM((C_PAD, L), jnp.float32),
            pltpu.SemaphoreType.DMA,
            pltpu.SemaphoreType.DMA,
            pltpu.SemaphoreType.DMA,
            pltpu.SemaphoreType.DMA,
        ],
        compiler_params=pltpu.CompilerParams(needs_layout_passes=False),
    )
    return f(x, y)


def _fin_body(ts_ref, tq_ref, c_ref, ss_ref, sq_ref, mean_ref, std_ref, valid_ref):
    ssc = jnp.sum(ss_ref[0], axis=0)[:, :D_SC]
    qsc = jnp.sum(sq_ref[0], axis=0)[:, :D_SC]
    s = jnp.concatenate([ts_ref[0], ssc], axis=-1)
    s2 = jnp.concatenate([tq_ref[0], qsc], axis=-1)
    cnt = c_ref[0]
    safe = jnp.maximum(cnt, 1.0)
    mean = s / safe
    denom = jnp.maximum(cnt - 1.0, 1.0)
    var = jnp.maximum((s2 - safe * mean * mean) / denom, 0.0)
    std = jnp.sqrt(var) + EPS
    v = cnt > float(COUNT)
    mean_ref[0] = jnp.where(v, mean, 0.0)[:NUM_CLASSES]
    std_ref[0] = jnp.where(v, std, 0.0)[:NUM_CLASSES]
    valid_ref[0] = v.astype(jnp.float32)


def kernel(x, y):
    B, d, N = x.shape
    tc_s, tc_q, tc_c = _tc_partial(x, y)
    sc_s, sc_q = _sc_partial(x, y)
    out_mean, out_std, out_valid = pl.pallas_call(
        _fin_body,
        grid=(B,),
        in_specs=[
            pl.BlockSpec((1, C_PAD, D_TC), lambda b: (b, 0, 0)),
            pl.BlockSpec((1, C_PAD, D_TC), lambda b: (b, 0, 0)),
            pl.BlockSpec((1, C_PAD, 1), lambda b: (b, 0, 0)),
            pl.BlockSpec((1, NP, C_PAD, L), lambda b: (b, 0, 0, 0)),
            pl.BlockSpec((1, NP, C_PAD, L), lambda b: (b, 0, 0, 0)),
        ],
        out_specs=[
            pl.BlockSpec((1, NUM_CLASSES, d), lambda b: (b, 0, 0)),
            pl.BlockSpec((1, NUM_CLASSES, d), lambda b: (b, 0, 0)),
            pl.BlockSpec((1, C_PAD, 1), lambda b: (b, 0, 0)),
        ],
        out_shape=[
            jax.ShapeDtypeStruct((B, NUM_CLASSES, d), jnp.float32),
            jax.ShapeDtypeStruct((B, NUM_CLASSES, d), jnp.float32),
            jax.ShapeDtypeStruct((B, C_PAD, 1), jnp.float32),
        ],
    )(tc_s, tc_q, tc_c, sc_s, sc_q)
    valid = out_valid[:, :NUM_CLASSES, 0] > 0.5
    return (out_mean, out_std, valid)


# speedup vs baseline: 6.1546x; 1.0557x over previous
import jax
import jax.numpy as jnp
from jax import lax
from jax.experimental import pallas as pl
from jax.experimental.pallas import tpu as pltpu
from jax.experimental.pallas import tpu_sc as plsc

COUNT = 6
EPS = 1e-05
NUM_CLASSES = 19
C_PAD = 24

B_SC = 1
NQ = 4
NPP = 8
DL = 16
CH = 2048
L = 16
ROW = NUM_CLASSES * L


def _tc_body(x_ref, y_ref, sum_ref, sq_ref, cnt_ref):
    b_i = pl.program_id(0)
    xb = x_ref[0]
    lab = y_ref[pl.ds(b_i, 1), :]
    cls = jax.lax.broadcasted_iota(jnp.int32, (C_PAD, xb.shape[1]), 0)
    oh = (cls == lab).astype(jnp.float32)
    dn = (((1,), (1,)), ((), ()))
    sum_ref[0] = jax.lax.dot_general(oh, xb, dn, preferred_element_type=jnp.float32)
    sq_ref[0] = jax.lax.dot_general(oh, xb * xb, dn, preferred_element_type=jnp.float32)
    cnt_ref[0] = jnp.sum(oh, axis=1, keepdims=True)


def _tc_partial(x, y):
    B, d, N = x.shape
    BT = B - B_SC
    return pl.pallas_call(
        _tc_body,
        grid=(BT,),
        in_specs=[
            pl.BlockSpec((1, d, N), lambda b: (b, 0, 0)),
            pl.BlockSpec((B, N), lambda b: (0, 0)),
        ],
        out_specs=[
            pl.BlockSpec((1, C_PAD, d), lambda b: (b, 0, 0)),
            pl.BlockSpec((1, C_PAD, d), lambda b: (b, 0, 0)),
            pl.BlockSpec((1, C_PAD, 1), lambda b: (b, 0, 0)),
        ],
        out_shape=[
            jax.ShapeDtypeStruct((BT, C_PAD, d), jnp.float32),
            jax.ShapeDtypeStruct((BT, C_PAD, d), jnp.float32),
            jax.ShapeDtypeStruct((BT, C_PAD, 1), jnp.float32),
        ],
    )(x, y)


def _sc_body(x_hbm, y_hbm, sum_out, sq_out, cnt_out,
             xb0, xb1, lb0, lb1, sum_e, sum_o, sq_e, sq_o, cnt_e, cnt_o,
             fin_s, fin_q, fin_c, sx0, sx1, sy0, sy1):
    wid = lax.axis_index("s") * 2 + lax.axis_index("c")
    q = wid % NQ
    pp = wid // NQ
    d0 = q * DL
    b = x_hbm.shape[0] - B_SC
    N = x_hbm.shape[2]
    npix = N // NPP
    base = pp * npix
    nch = npix // CH

    zeros = jnp.zeros((L,), jnp.float32)
    ones = jnp.ones((L,), jnp.float32)
    lane = lax.broadcasted_iota(jnp.int32, (L,), 0)

    def init_body(i, _):
        sum_e[pl.ds(i * L, L)] = zeros
        sum_o[pl.ds(i * L, L)] = zeros
        sq_e[pl.ds(i * L, L)] = zeros
        sq_o[pl.ds(i * L, L)] = zeros
        return 0

    lax.fori_loop(0, L * NUM_CLASSES, init_body, 0)

    def init_cnt(i, _):
        cnt_e[pl.ds(i * L, L)] = zeros
        cnt_o[pl.ds(i * L, L)] = zeros
        return 0

    lax.fori_loop(0, 2 * L, init_cnt, 0)

    xbufs = (xb0, xb1)
    lbufs = (lb0, lb1)
    xsems = (sx0, sx1)
    ysems = (sy0, sy1)

    def start(t):
        s = t % 2
        hx = pltpu.async_copy(
            x_hbm.at[b, pl.ds(d0, DL), pl.ds(base + t * CH, CH)],
            xbufs[s], xsems[s])
        hy = pltpu.async_copy(
            y_hbm.at[b, pl.ds(base + t * CH, CH)], lbufs[s], ysems[s])
        return hx, hy

    hands = start(0)
    for t in range(nch):
        s = t % 2
        nxt = start(t + 1) if t + 1 < nch else None
        hands[0].wait()
        hands[1].wait()
        xbuf = xbufs[s]
        lbuf = lbufs[s]

        def group_body(i):
            g0 = i * 2
            lv0 = lbuf[pl.ds(g0 * L, L)]
            lv1 = lbuf[pl.ds(g0 * L + L, L)]
            idx0 = lv0 * L + lane
            idx1 = lv1 * L + lane
            plsc.addupdate_scatter(cnt_e, [idx0], ones)
            plsc.addupdate_scatter(cnt_o, [idx1], ones)
            for dl in range(DL):
                v0 = xbuf[dl, pl.ds(g0 * L, L)]
                v1 = xbuf[dl, pl.ds(g0 * L + L, L)]
                ie = idx0 + dl * ROW
                io = idx1 + dl * ROW
                plsc.addupdate_scatter(sum_e, [ie], v0)
                plsc.addupdate_scatter(sum_o, [io], v1)
                plsc.addupdate_scatter(sq_e, [ie], v0 * v0)
                plsc.addupdate_scatter(sq_o, [io], v1 * v1)

        plsc.parallel_loop(0, CH // (2 * L), unroll=2)(group_body)
        hands = nxt

    lane_row = lane * ROW
    for c in range(C_PAD):
        if c < NUM_CLASSES:
            vs = zeros
            vq = zeros
            for l in range(L):
                idx = lane_row + (c * L + l)
                vs = vs + plsc.load_gather(sum_e, [idx]) + plsc.load_gather(sum_o, [idx])
                vq = vq + plsc.load_gather(sq_e, [idx]) + plsc.load_gather(sq_o, [idx])
            fin_s[c, :] = vs
            fin_q[c, :] = vq
        else:
            fin_s[c, :] = zeros
            fin_q[c, :] = zeros

    pltpu.sync_copy(fin_s, sum_out.at[q, pp])
    pltpu.sync_copy(fin_q, sq_out.at[q, pp])

    for h in range(2):
        vc = zeros
        for l in range(L):
            idx = (h * L + lane) * L + l
            vc = vc + plsc.load_gather(cnt_e, [idx]) + plsc.load_gather(cnt_o, [idx])
        fin_c[pl.ds(h * L, L)] = vc

    @pl.when(q == 0)
    def _():
        pltpu.sync_copy(fin_c, cnt_out.at[pp])


def _sc_partial(x, y):
    B, d, N = x.shape
    mesh = plsc.VectorSubcoreMesh(core_axis_name="c", subcore_axis_name="s")
    f = pl.kernel(
        _sc_body,
        out_type=[
            jax.ShapeDtypeStruct((NQ, NPP, C_PAD, L), jnp.float32),
            jax.ShapeDtypeStruct((NQ, NPP, C_PAD, L), jnp.float32),
            jax.ShapeDtypeStruct((NPP, 2 * L), jnp.float32),
        ],
        mesh=mesh,
        scratch_types=[
            pltpu.VMEM((DL, CH), jnp.float32),
            pltpu.VMEM((DL, CH), jnp.float32),
            pltpu.VMEM((CH,), jnp.int32),
            pltpu.VMEM((CH,), jnp.int32),
            pltpu.VMEM((L * NUM_CLASSES * L,), jnp.float32),
            pltpu.VMEM((L * NUM_CLASSES * L,), jnp.float32),
            pltpu.VMEM((L * NUM_CLASSES * L,), jnp.float32),
            pltpu.VMEM((L * NUM_CLASSES * L,), jnp.float32),
            pltpu.VMEM((2 * L * L,), jnp.float32),
            pltpu.VMEM((2 * L * L,), jnp.float32),
            pltpu.VMEM((C_PAD, L), jnp.float32),
            pltpu.VMEM((C_PAD, L), jnp.float32),
            pltpu.VMEM((2 * L,), jnp.float32),
            pltpu.SemaphoreType.DMA,
            pltpu.SemaphoreType.DMA,
            pltpu.SemaphoreType.DMA,
            pltpu.SemaphoreType.DMA,
        ],
        compiler_params=pltpu.CompilerParams(needs_layout_passes=False),
    )
    return f(x, y)


def _fin_body(ts_ref, tq_ref, tc_ref, ss_ref, sq_ref, sc_ref,
              mean_ref, std_ref, valid_ref):
    s4 = jnp.sum(ss_ref[...], axis=1)
    q4 = jnp.sum(sq_ref[...], axis=1)
    s7 = jnp.concatenate([s4[i] for i in range(NQ)], axis=-1)
    q7 = jnp.concatenate([q4[i] for i in range(NQ)], axis=-1)
    c7 = jnp.sum(sc_ref[...], axis=0)[:C_PAD, None]
    s = jnp.concatenate([ts_ref[...], s7[None]], axis=0)
    s2 = jnp.concatenate([tq_ref[...], q7[None]], axis=0)
    cnt = jnp.concatenate([tc_ref[...], c7[None]], axis=0)
    safe = jnp.maximum(cnt, 1.0)
    mean = s / safe
    denom = jnp.maximum(cnt - 1.0, 1.0)
    var = jnp.maximum((s2 - safe * mean * mean) / denom, 0.0)
    std = jnp.sqrt(var) + EPS
    v = cnt > float(COUNT)
    mean_ref[...] = jnp.where(v, mean, 0.0)[:, :NUM_CLASSES]
    std_ref[...] = jnp.where(v, std, 0.0)[:, :NUM_CLASSES]
    valid_ref[...] = v.astype(jnp.float32)


def kernel(x, y):
    B, d, N = x.shape
    BT = B - B_SC
    tc_s, tc_q, tc_c = _tc_partial(x, y)
    sc_s, sc_q, sc_c = _sc_partial(x, y)
    out_mean, out_std, out_valid = pl.pallas_call(
        _fin_body,
        grid=(1,),
        in_specs=[
            pl.BlockSpec((BT, C_PAD, d), lambda i: (0, 0, 0)),
            pl.BlockSpec((BT, C_PAD, d), lambda i: (0, 0, 0)),
            pl.BlockSpec((BT, C_PAD, 1), lambda i: (0, 0, 0)),
            pl.BlockSpec((NQ, NPP, C_PAD, L), lambda i: (0, 0, 0, 0)),
            pl.BlockSpec((NQ, NPP, C_PAD, L), lambda i: (0, 0, 0, 0)),
            pl.BlockSpec((NPP, 2 * L), lambda i: (0, 0)),
        ],
        out_specs=[
            pl.BlockSpec((B, NUM_CLASSES, d), lambda i: (0, 0, 0)),
            pl.BlockSpec((B, NUM_CLASSES, d), lambda i: (0, 0, 0)),
            pl.BlockSpec((B, C_PAD, 1), lambda i: (0, 0, 0)),
        ],
        out_shape=[
            jax.ShapeDtypeStruct((B, NUM_CLASSES, d), jnp.float32),
            jax.ShapeDtypeStruct((B, NUM_CLASSES, d), jnp.float32),
            jax.ShapeDtypeStruct((B, C_PAD, 1), jnp.float32),
        ],
    )(tc_s, tc_q, tc_c, sc_s, sc_q, sc_c)
    valid = out_valid[:, :NUM_CLASSES, 0] > 0.5
    return (out_mean, out_std, valid)


# speedup vs baseline: 6.1747x; 1.0033x over previous
import jax
import jax.numpy as jnp
from jax import lax
from jax.experimental import pallas as pl
from jax.experimental.pallas import tpu as pltpu
from jax.experimental.pallas import tpu_sc as plsc

COUNT = 6
EPS = 1e-05
NUM_CLASSES = 19
C_PAD = 24

B_SC = 1
NQ = 4
NPP = 8
DL = 16
CH = 2048
L = 16
ROW = NUM_CLASSES * L


def _tc_body(x_ref, y_ref, sum_ref, sq_ref, cnt_ref):
    b_i = pl.program_id(0)
    xb = x_ref[0]
    lab = y_ref[pl.ds(b_i, 1), :]
    cls = jax.lax.broadcasted_iota(jnp.int32, (C_PAD, xb.shape[1]), 0)
    oh = (cls == lab).astype(jnp.float32)
    dn = (((1,), (1,)), ((), ()))
    sum_ref[0] = jax.lax.dot_general(oh, xb, dn, preferred_element_type=jnp.float32)
    sq_ref[0] = jax.lax.dot_general(oh, xb * xb, dn, preferred_element_type=jnp.float32)
    cnt_ref[0] = jnp.sum(oh, axis=1, keepdims=True)


def _tc_partial(x, y):
    B, d, N = x.shape
    BT = B - B_SC
    return pl.pallas_call(
        _tc_body,
        grid=(BT,),
        in_specs=[
            pl.BlockSpec((1, d, N), lambda b: (b, 0, 0)),
            pl.BlockSpec((B, N), lambda b: (0, 0)),
        ],
        out_specs=[
            pl.BlockSpec((1, C_PAD, d), lambda b: (b, 0, 0)),
            pl.BlockSpec((1, C_PAD, d), lambda b: (b, 0, 0)),
            pl.BlockSpec((1, C_PAD, 1), lambda b: (b, 0, 0)),
        ],
        out_shape=[
            jax.ShapeDtypeStruct((BT, C_PAD, d), jnp.float32),
            jax.ShapeDtypeStruct((BT, C_PAD, d), jnp.float32),
            jax.ShapeDtypeStruct((BT, C_PAD, 1), jnp.float32),
        ],
    )(x, y)


def _sc_body(x_hbm, y_hbm, sum_out, sq_out, cnt_out,
             xb0, xb1, lb0, lb1, sum_e, sum_o, sq_e, sq_o, cnt_e, cnt_o,
             fin_s, fin_q, fin_c, sx0, sx1, sy0, sy1):
    wid = lax.axis_index("s") * 2 + lax.axis_index("c")
    q = wid % NQ
    pp = wid // NQ
    d0 = q * DL
    b = x_hbm.shape[0] - B_SC
    N = x_hbm.shape[2]
    npix = N // NPP
    base = pp * npix
    nch = npix // CH

    zeros = jnp.zeros((L,), jnp.float32)
    ones = jnp.ones((L,), jnp.float32)
    lane = lax.broadcasted_iota(jnp.int32, (L,), 0)

    def init_body(i, _):
        sum_e[pl.ds(i * L, L)] = zeros
        sum_o[pl.ds(i * L, L)] = zeros
        sq_e[pl.ds(i * L, L)] = zeros
        sq_o[pl.ds(i * L, L)] = zeros
        return 0

    lax.fori_loop(0, L * NUM_CLASSES, init_body, 0)

    def init_cnt(i, _):
        cnt_e[pl.ds(i * L, L)] = zeros
        cnt_o[pl.ds(i * L, L)] = zeros
        return 0

    lax.fori_loop(0, 2 * L, init_cnt, 0)

    xbufs = (xb0, xb1)
    lbufs = (lb0, lb1)
    xsems = (sx0, sx1)
    ysems = (sy0, sy1)

    def start(t):
        s = t % 2
        hx = pltpu.async_copy(
            x_hbm.at[b, pl.ds(d0, DL), pl.ds(base + t * CH, CH)],
            xbufs[s], xsems[s])
        hy = pltpu.async_copy(
            y_hbm.at[b, pl.ds(base + t * CH, CH)], lbufs[s], ysems[s])
        return hx, hy

    hands = start(0)
    for t in range(nch):
        s = t % 2
        nxt = start(t + 1) if t + 1 < nch else None
        hands[0].wait()
        hands[1].wait()
        xbuf = xbufs[s]
        lbuf = lbufs[s]

        def group_body(i):
            g0 = i * 2
            lv0 = lbuf[pl.ds(g0 * L, L)]
            lv1 = lbuf[pl.ds(g0 * L + L, L)]
            idx0 = lv0 * L + lane
            idx1 = lv1 * L + lane
            plsc.addupdate_scatter(cnt_e, [idx0], ones)
            plsc.addupdate_scatter(cnt_o, [idx1], ones)
            for dl in range(DL):
                v0 = xbuf[dl, pl.ds(g0 * L, L)]
                v1 = xbuf[dl, pl.ds(g0 * L + L, L)]
                ie = idx0 + dl * ROW
                io = idx1 + dl * ROW
                plsc.addupdate_scatter(sum_e, [ie], v0)
                plsc.addupdate_scatter(sum_o, [io], v1)
                plsc.addupdate_scatter(sq_e, [ie], v0 * v0)
                plsc.addupdate_scatter(sq_o, [io], v1 * v1)

        plsc.parallel_loop(0, CH // (2 * L), unroll=2)(group_body)
        hands = nxt

    lane_row = lane * ROW
    for c in range(C_PAD):
        if c < NUM_CLASSES:
            vs = zeros
            vq = zeros
            for l in range(L):
                idx = lane_row + (c * L + l)
                vs = vs + plsc.load_gather(sum_e, [idx]) + plsc.load_gather(sum_o, [idx])
                vq = vq + plsc.load_gather(sq_e, [idx]) + plsc.load_gather(sq_o, [idx])
            fin_s[c, :] = vs
            fin_q[c, :] = vq
        else:
            fin_s[c, :] = zeros
            fin_q[c, :] = zeros

    pltpu.sync_copy(fin_s, sum_out.at[q, pp])
    pltpu.sync_copy(fin_q, sq_out.at[q, pp])

    for h in range(2):
        vc = zeros
        for l in range(L):
            idx = (h * L + lane) * L + l
            vc = vc + plsc.load_gather(cnt_e, [idx]) + plsc.load_gather(cnt_o, [idx])
        fin_c[pl.ds(h * L, L)] = vc

    @pl.when(q == 0)
    def _():
        pltpu.sync_copy(fin_c, cnt_out.at[pp])


def _sc_partial(x, y):
    B, d, N = x.shape
    mesh = plsc.VectorSubcoreMesh(core_axis_name="c", subcore_axis_name="s")
    f = pl.kernel(
        _sc_body,
        out_type=[
            jax.ShapeDtypeStruct((NQ, NPP, C_PAD, L), jnp.float32),
            jax.ShapeDtypeStruct((NQ, NPP, C_PAD, L), jnp.float32),
            jax.ShapeDtypeStruct((NPP, 2 * L), jnp.float32),
        ],
        mesh=mesh,
        scratch_types=[
            pltpu.VMEM((DL, CH), jnp.float32),
            pltpu.VMEM((DL, CH), jnp.float32),
            pltpu.VMEM((CH,), jnp.int32),
            pltpu.VMEM((CH,), jnp.int32),
            pltpu.VMEM((L * NUM_CLASSES * L,), jnp.float32),
            pltpu.VMEM((L * NUM_CLASSES * L,), jnp.float32),
            pltpu.VMEM((L * NUM_CLASSES * L,), jnp.float32),
            pltpu.VMEM((L * NUM_CLASSES * L,), jnp.float32),
            pltpu.VMEM((2 * L * L,), jnp.float32),
            pltpu.VMEM((2 * L * L,), jnp.float32),
            pltpu.VMEM((C_PAD, L), jnp.float32),
            pltpu.VMEM((C_PAD, L), jnp.float32),
            pltpu.VMEM((2 * L,), jnp.float32),
            pltpu.SemaphoreType.DMA,
            pltpu.SemaphoreType.DMA,
            pltpu.SemaphoreType.DMA,
            pltpu.SemaphoreType.DMA,
        ],
        compiler_params=pltpu.CompilerParams(needs_layout_passes=False),
    )
    return f(x, y)


def _fin_body(ts_ref, tq_ref, tc_ref, ss_ref, sq_ref, sc_ref,
              mean_ref, std_ref, valid_ref):
    s4 = jnp.sum(ss_ref[...], axis=1)
    q4 = jnp.sum(sq_ref[...], axis=1)
    s7 = jnp.concatenate([s4[i] for i in range(NQ)], axis=-1)
    q7 = jnp.concatenate([q4[i] for i in range(NQ)], axis=-1)
    c7 = jnp.sum(sc_ref[...], axis=0)[:C_PAD, None]
    s = jnp.concatenate([ts_ref[...], s7[None]], axis=0)
    s2 = jnp.concatenate([tq_ref[...], q7[None]], axis=0)
    cnt = jnp.concatenate([tc_ref[...], c7[None]], axis=0)
    safe = jnp.maximum(cnt, 1.0)
    mean = s / safe
    denom = jnp.maximum(cnt - 1.0, 1.0)
    var = jnp.maximum((s2 - safe * mean * mean) / denom, 0.0)
    std = jnp.sqrt(var) + EPS
    v = cnt > float(COUNT)
    mean_ref[...] = jnp.where(v, mean, 0.0)[:, :NUM_CLASSES]
    std_ref[...] = jnp.where(v, std, 0.0)[:, :NUM_CLASSES]
    valid_ref[...] = v[:, :NUM_CLASSES, 0][:, None, :]


def kernel(x, y):
    B, d, N = x.shape
    BT = B - B_SC
    tc_s, tc_q, tc_c = _tc_partial(x, y)
    sc_s, sc_q, sc_c = _sc_partial(x, y)
    out_mean, out_std, out_valid = pl.pallas_call(
        _fin_body,
        grid=(1,),
        in_specs=[
            pl.BlockSpec((BT, C_PAD, d), lambda i: (0, 0, 0)),
            pl.BlockSpec((BT, C_PAD, d), lambda i: (0, 0, 0)),
            pl.BlockSpec((BT, C_PAD, 1), lambda i: (0, 0, 0)),
            pl.BlockSpec((NQ, NPP, C_PAD, L), lambda i: (0, 0, 0, 0)),
            pl.BlockSpec((NQ, NPP, C_PAD, L), lambda i: (0, 0, 0, 0)),
            pl.BlockSpec((NPP, 2 * L), lambda i: (0, 0)),
        ],
        out_specs=[
            pl.BlockSpec((B, NUM_CLASSES, d), lambda i: (0, 0, 0)),
            pl.BlockSpec((B, NUM_CLASSES, d), lambda i: (0, 0, 0)),
            pl.BlockSpec((B, 1, NUM_CLASSES), lambda i: (0, 0, 0)),
        ],
        out_shape=[
            jax.ShapeDtypeStruct((B, NUM_CLASSES, d), jnp.float32),
            jax.ShapeDtypeStruct((B, NUM_CLASSES, d), jnp.float32),
            jax.ShapeDtypeStruct((B, 1, NUM_CLASSES), jnp.bool_),
        ],
    )(tc_s, tc_q, tc_c, sc_s, sc_q, sc_c)
    return (out_mean, out_std, out_valid[:, 0, :])
